# baseline (speedup 1.0000x reference)
import functools

import jax
import jax.numpy as jnp
from jax import lax
from jax.experimental import pallas as pl
from jax.experimental.pallas import tpu as pltpu
from jax.experimental.pallas import tpu_sc as plsc

O = 10000
E = 160000
DIN = 128
H = 128
DOUT = 128
EPS = 1e-3

NC = 2
NS = 16
NW = NC * NS
LANES = 16

CHUNK_G = 64
NCHUNK_G = E // CHUNK_G
BASE_G = NCHUNK_G // NW
EXTRA_G = NCHUNK_G - BASE_G * NW
CHUNK_S = 32
ROWS_S = 2 * CHUNK_S
NCHUNK_S = E // CHUNK_S
BASE_S = NCHUNK_S // NW
EXTRA_S = NCHUNK_S - BASE_S * NW
HP = H + LANES
O_PAD = 10240
ROWS_PER_SUB = O_PAD // NS

@functools.cache
def _sc_mesh():
  return plsc.VectorSubcoreMesh(
      core_axis_name="c", subcore_axis_name="s", num_cores=NC,
      num_subcores=NS)


def _lrelu(x):
  return jnp.where(x >= 0, x, 0.2 * x)


def _proj_nodes_body(obj_ref, wa_ref, wc_ref, pa_ref, pc_ref):
  x = obj_ref[...]
  pa_ref[...] = jnp.dot(x, wa_ref[...], preferred_element_type=jnp.float32)
  pc_ref[...] = jnp.dot(x, wc_ref[...], preferred_element_type=jnp.float32)


def _proj_nodes(obj_vecs, w1a, w1c):
  bo = 2000
  return pl.pallas_call(
      _proj_nodes_body,
      grid=(O // bo,),
      in_specs=[
          pl.BlockSpec((bo, DIN), lambda i: (i, 0)),
          pl.BlockSpec((DIN, H), lambda i: (0, 0)),
          pl.BlockSpec((DIN, H), lambda i: (0, 0)),
      ],
      out_specs=[
          pl.BlockSpec((bo, H), lambda i: (i, 0)),
          pl.BlockSpec((bo, H), lambda i: (i, 0)),
      ],
      out_shape=[
          jax.ShapeDtypeStruct((O, H), jnp.float32),
          jax.ShapeDtypeStruct((O, H), jnp.float32),
      ],
  )(obj_vecs, w1a, w1c)


def _proj_pred_body(pred_ref, wb_ref, b1_ref, out_ref):
  out_ref[...] = (
      jnp.dot(pred_ref[...], wb_ref[...], preferred_element_type=jnp.float32)
      + b1_ref[...])


def _proj_pred(pred_vecs, w1b, b1r):
  be = 2000
  return pl.pallas_call(
      _proj_pred_body,
      grid=(E // be,),
      in_specs=[
          pl.BlockSpec((be, DIN), lambda i: (i, 0)),
          pl.BlockSpec((DIN, H), lambda i: (0, 0)),
          pl.BlockSpec((1, H), lambda i: (0, 0)),
      ],
      out_specs=pl.BlockSpec((be, H), lambda i: (i, 0)),
      out_shape=jax.ShapeDtypeStruct((E, H), jnp.float32),
  )(pred_vecs, w1b, b1r)


def _sc_gather_body(pa_hbm, pc_hbm, ppred_hbm, sidx_hbm, oidx_hbm,
                    hpre_hbm,
                    idx_s, idx_o, buf_s, buf_o, buf_p, sem_g, sem_st):
  c = lax.axis_index("c")
  s = lax.axis_index("s")
  wid = c * NS + s
  trips = BASE_G + jnp.where(wid < EXTRA_G, 1, 0)

  def chunk_body(i, _):
    base = (i * NW + wid) * CHUNK_G
    pltpu.sync_copy(sidx_hbm.at[pl.ds(base, CHUNK_G)], idx_s)
    pltpu.sync_copy(oidx_hbm.at[pl.ds(base, CHUNK_G)], idx_o)
    cp_s = pltpu.async_copy(pa_hbm.at[idx_s], buf_s, sem_g)
    cp_o = pltpu.async_copy(pc_hbm.at[idx_o], buf_o, sem_g)

    @pl.when(i > 0)
    def _():
      prev = ((i - 1) * NW + wid) * CHUNK_G
      pltpu.make_async_copy(buf_p, hpre_hbm.at[pl.ds(prev, CHUNK_G)],
                            sem_st).wait()

    pltpu.async_copy(ppred_hbm.at[pl.ds(base, CHUNK_G)], buf_p, sem_g).wait()
    cp_s.wait()
    cp_o.wait()

    def row_body(r, carry):
      for j in range(8):
        sl = pl.ds(j * LANES, LANES)
        buf_p[r, sl] = buf_s[r, sl] + buf_o[r, sl] + buf_p[r, sl]
      return carry

    lax.fori_loop(0, CHUNK_G, row_body, 0)
    pltpu.async_copy(buf_p, hpre_hbm.at[pl.ds(base, CHUNK_G)], sem_st)
    return 0

  lax.fori_loop(0, trips, chunk_body, 0)
  last = ((trips - 1) * NW + wid) * CHUNK_G
  pltpu.make_async_copy(buf_p, hpre_hbm.at[pl.ds(last, CHUNK_G)],
                        sem_st).wait()


@functools.cache
def _sc_gather_kernel():
  return pl.kernel(
      _sc_gather_body,
      out_type=jax.ShapeDtypeStruct((E, H), jnp.float32),
      mesh=_sc_mesh(),
      scratch_types=[
          pltpu.VMEM((CHUNK_G,), jnp.int32),
          pltpu.VMEM((CHUNK_G,), jnp.int32),
          pltpu.VMEM((CHUNK_G, H), jnp.float32),
          pltpu.VMEM((CHUNK_G, H), jnp.float32),
          pltpu.VMEM((CHUNK_G, H), jnp.float32),
          pltpu.SemaphoreType.DMA,
          pltpu.SemaphoreType.DMA,
      ],
  )


def _sc_gather(*args):
  return _sc_gather_kernel()(*args)


def _bn1_affine_body(hpre_ref, g1_ref, be1_ref, scale_ref, shift_ref,
                     acc_s, acc_q):
  i = pl.program_id(0)

  @pl.when(i == 0)
  def _():
    acc_s[...] = jnp.zeros_like(acc_s)
    acc_q[...] = jnp.zeros_like(acc_q)

  x = hpre_ref[...]
  acc_s[...] += jnp.sum(x, axis=0, keepdims=True)
  acc_q[...] += jnp.sum(x * x, axis=0, keepdims=True)

  @pl.when(i == pl.num_programs(0) - 1)
  def _():
    m = acc_s[...] / E
    v = acc_q[...] / E - m * m
    scale = g1_ref[...] * lax.rsqrt(v + EPS)
    scale_ref[...] = scale
    shift_ref[...] = be1_ref[...] - m * scale


def _bn1_affine(hpre, g1r, be1r):
  be = 4000
  return pl.pallas_call(
      _bn1_affine_body,
      grid=(E // be,),
      in_specs=[
          pl.BlockSpec((be, H), lambda i: (i, 0)),
          pl.BlockSpec((1, H), lambda i: (0, 0)),
          pl.BlockSpec((1, H), lambda i: (0, 0)),
      ],
      out_specs=[
          pl.BlockSpec((1, H), lambda i: (0, 0)),
          pl.BlockSpec((1, H), lambda i: (0, 0)),
      ],
      out_shape=[
          jax.ShapeDtypeStruct((1, H), jnp.float32),
          jax.ShapeDtypeStruct((1, H), jnp.float32),
      ],
      scratch_shapes=[
          pltpu.VMEM((1, H), jnp.float32),
          pltpu.VMEM((1, H), jnp.float32),
      ],
  )(hpre, g1r, be1r)


def _gram_body(hpre_ref, scale_ref, shift_ref, sum_ref, gram_ref,
               acc_s, acc_g):
  i = pl.program_id(0)

  @pl.when(i == 0)
  def _():
    acc_s[...] = jnp.zeros_like(acc_s)
    acc_g[...] = jnp.zeros_like(acc_g)

  h = _lrelu(hpre_ref[...] * scale_ref[...] + shift_ref[...])
  acc_s[...] += jnp.sum(h, axis=0, keepdims=True)
  acc_g[...] += lax.dot_general(h, h, (((0,), (0,)), ((), ())),
                                preferred_element_type=jnp.float32)

  @pl.when(i == pl.num_programs(0) - 1)
  def _():
    sum_ref[...] = acc_s[...]
    gram_ref[...] = acc_g[...]


def _gram(hpre, scale1, shift1):
  be = 2000
  return pl.pallas_call(
      _gram_body,
      grid=(E // be,),
      in_specs=[
          pl.BlockSpec((be, H), lambda i: (i, 0)),
          pl.BlockSpec((1, H), lambda i: (0, 0)),
          pl.BlockSpec((1, H), lambda i: (0, 0)),
      ],
      out_specs=[
          pl.BlockSpec((1, H), lambda i: (0, 0)),
          pl.BlockSpec((H, H), lambda i: (0, 0)),
      ],
      out_shape=[
          jax.ShapeDtypeStruct((1, H), jnp.float32),
          jax.ShapeDtypeStruct((H, H), jnp.float32),
      ],
      scratch_shapes=[
          pltpu.VMEM((1, H), jnp.float32),
          pltpu.VMEM((H, H), jnp.float32),
      ],
  )(hpre, scale1, shift1)


def _bn2_affine_body(sum_ref, gram_ref, w2_ref, b2_ref, g2_ref, be2_ref,
                     scale_ref, shift_ref):
  mh = sum_ref[...] / E
  cov = gram_ref[...] / E - mh.T * mh
  w2 = w2_ref[...]
  cw = jnp.dot(cov, w2, preferred_element_type=jnp.float32)
  v2 = jnp.sum(w2 * cw, axis=0, keepdims=True)
  m2 = jnp.dot(mh, w2, preferred_element_type=jnp.float32) + b2_ref[...]
  scale = g2_ref[...] * lax.rsqrt(v2 + EPS)
  scale_ref[...] = scale
  shift_ref[...] = be2_ref[...] - m2 * scale


def _bn2_affine(sumh, gram, w2, b2r, g2r, be2r):
  w = 2 * H + DOUT
  return pl.pallas_call(
      _bn2_affine_body,
      out_shape=[
          jax.ShapeDtypeStruct((1, w), jnp.float32),
          jax.ShapeDtypeStruct((1, w), jnp.float32),
      ],
  )(sumh, gram, w2, b2r, g2r, be2r)


def _edge_out_body(hpre_ref, sc1_ref, sh1_ref, w2_ref, sc2_ref, sh2_ref,
                   pay_ref, np_ref):
  h = _lrelu(hpre_ref[...] * sc1_ref[...] + sh1_ref[...])
  t2 = jnp.dot(h, w2_ref[...], preferred_element_type=jnp.float32)
  u = _lrelu(t2 * sc2_ref[...] + sh2_ref[...])
  n = u.shape[0]
  ones = jnp.ones((n, LANES), jnp.float32)
  s_rows = jnp.concatenate([u[:, :H], ones], axis=1)
  o_rows = jnp.concatenate([u[:, H + DOUT:], ones], axis=1)
  nb = n // CHUNK_S
  pay = jnp.concatenate([s_rows.reshape(nb, CHUNK_S, HP),
                         o_rows.reshape(nb, CHUNK_S, HP)], axis=1)
  pay_ref[...] = pay
  np_ref[...] = u[:, H:H + DOUT]


def _edge_out(hpre, scale1, shift1, w2, scale2, shift2):
  be = 1600
  nb = be // CHUNK_S
  w = 2 * H + DOUT
  return pl.pallas_call(
      _edge_out_body,
      grid=(E // be,),
      in_specs=[
          pl.BlockSpec((be, H), lambda i: (i, 0)),
          pl.BlockSpec((1, H), lambda i: (0, 0)),
          pl.BlockSpec((1, H), lambda i: (0, 0)),
          pl.BlockSpec((H, w), lambda i: (0, 0)),
          pl.BlockSpec((1, w), lambda i: (0, 0)),
          pl.BlockSpec((1, w), lambda i: (0, 0)),
      ],
      out_specs=[
          pl.BlockSpec((nb, ROWS_S, HP), lambda i: (i, 0, 0)),
          pl.BlockSpec((be, DOUT), lambda i: (i, 0)),
      ],
      out_shape=[
          jax.ShapeDtypeStruct((NCHUNK_S, ROWS_S, HP), jnp.float32),
          jax.ShapeDtypeStruct((E, DOUT), jnp.float32),
      ],
  )(hpre, scale1, shift1, w2, scale2, shift2)


def _sc_scatter_body(pay_hbm, midx_hbm,
                     pooled_hbm,
                     pooled_sh, idx_v, buf_v, sem_d, sem_sc):
  c = lax.axis_index("c")
  s = lax.axis_index("s")
  wid = c * NS + s
  trips = BASE_S + jnp.where(wid < EXTRA_S, 1, 0)

  zero16 = jnp.zeros((LANES,), jnp.float32)

  def zbuf_body(r, _):
    for j in range(HP // LANES):
      buf_v[r, pl.ds(j * LANES, LANES)] = zero16
    return 0

  lax.fori_loop(0, ROWS_S, zbuf_body, 0)

  n_blk = ROWS_PER_SUB // ROWS_S
  row0 = s * ROWS_PER_SUB

  def zero_body(t, _):
    pltpu.sync_copy(buf_v, pooled_sh.at[pl.ds(row0 + t * ROWS_S, ROWS_S)])
    return 0

  lax.fori_loop(0, n_blk, zero_body, 0)
  plsc.subcore_barrier()

  def chunk_body(i, _):
    g = i * NW + wid
    pltpu.sync_copy(midx_hbm.at[g], idx_v)
    pltpu.sync_copy(pay_hbm.at[g], buf_v)
    pltpu.sync_copy(buf_v, pooled_sh.at[idx_v], add=True)
    return 0

  lax.fori_loop(0, trips, chunk_body, 0)
  plsc.subcore_barrier()

  def dump_body(t, _):
    sl = pl.ds(row0 + t * ROWS_S, ROWS_S)
    pltpu.sync_copy(pooled_sh.at[sl], buf_v)
    pltpu.sync_copy(buf_v, pooled_hbm.at[c, sl])
    return 0

  lax.fori_loop(0, n_blk, dump_body, 0)


@functools.cache
def _sc_scatter_kernel():
  return pl.kernel(
      _sc_scatter_body,
      out_type=jax.ShapeDtypeStruct((NC, O_PAD, HP), jnp.float32),
      mesh=_sc_mesh(),
      compiler_params=pltpu.CompilerParams(use_tc_tiling_on_sc=False),
      scratch_types=[
          pltpu.VMEM_SHARED((O_PAD, HP), jnp.float32),
          pltpu.VMEM((ROWS_S,), jnp.int32),
          pltpu.VMEM((ROWS_S, HP), jnp.float32),
          pltpu.SemaphoreType.DMA,
          pltpu.SemaphoreType.DMA,
      ],
  )


def _sc_scatter(*args):
  return _sc_scatter_kernel()(*args)


def _node_mlp_body(pp_ref, w3_ref, b3_ref, g3_ref, be3_ref,
                   w4_ref, b4_ref, g4_ref, be4_ref, out_ref):
  acc = pp_ref[0, :O, :] + pp_ref[1, :O, :]
  pooled = acc[:, :H]
  cnt = jnp.clip(acc[:, H:H + 1], 1.0, float(O))
  pooled = pooled / cnt

  x = jnp.dot(pooled, w3_ref[...], preferred_element_type=jnp.float32)
  x = x + b3_ref[...]
  m = jnp.mean(x, axis=0, keepdims=True)
  v = jnp.mean((x - m) * (x - m), axis=0, keepdims=True)
  h2 = _lrelu((x - m) * lax.rsqrt(v + EPS) * g3_ref[...] + be3_ref[...])

  y = jnp.dot(h2, w4_ref[...], preferred_element_type=jnp.float32)
  y = y + b4_ref[...]
  m2 = jnp.mean(y, axis=0, keepdims=True)
  v2 = jnp.mean((y - m2) * (y - m2), axis=0, keepdims=True)
  out_ref[...] = _lrelu((y - m2) * lax.rsqrt(v2 + EPS) * g4_ref[...]
                        + be4_ref[...])


def _node_mlp(pooledp, w3, b3r, g3r, be3r, w4, b4r, g4r, be4r):
  return pl.pallas_call(
      _node_mlp_body,
      out_shape=jax.ShapeDtypeStruct((O, DOUT), jnp.float32),
  )(pooledp, w3, b3r, g3r, be3r, w4, b4r, g4r, be4r)


def kernel(obj_vecs, pred_vecs, edges, W1, b1, g1, be1, W2, b2, g2, be2,
           W3, b3, g3, be3, W4, b4, g4, be4):
  edges = edges.astype(jnp.int32)
  s_idx = edges[:, 0]
  o_idx = edges[:, 1]

  w1a = W1[0:DIN, :]
  w1b = W1[DIN:2 * DIN, :]
  w1c = W1[2 * DIN:3 * DIN, :]

  b1r = b1.reshape(1, H)
  g1r = g1.reshape(1, H)
  be1r = be1.reshape(1, H)
  w = 2 * H + DOUT
  b2r = b2.reshape(1, w)
  g2r = g2.reshape(1, w)
  be2r = be2.reshape(1, w)
  b3r = b3.reshape(1, H)
  g3r = g3.reshape(1, H)
  be3r = be3.reshape(1, H)
  b4r = b4.reshape(1, DOUT)
  g4r = g4.reshape(1, DOUT)
  be4r = be4.reshape(1, DOUT)

  pa, pc = _proj_nodes(obj_vecs, w1a, w1c)
  ppred = _proj_pred(pred_vecs, w1b, b1r)

  hpre = _sc_gather(pa, pc, ppred, s_idx, o_idx)

  scale1, shift1 = _bn1_affine(hpre, g1r, be1r)
  sumh, gram = _gram(hpre, scale1, shift1)
  scale2, shift2 = _bn2_affine(sumh, gram, W2, b2r, g2r, be2r)

  payload, new_p = _edge_out(hpre, scale1, shift1, W2, scale2, shift2)
  midx = jnp.concatenate([s_idx.reshape(NCHUNK_S, CHUNK_S),
                          o_idx.reshape(NCHUNK_S, CHUNK_S)], axis=1)

  pooledp = _sc_scatter(payload, midx)

  new_obj = _node_mlp(pooledp, W3, b3r, g3r, be3r, W4, b4r, g4r, be4r)
  return (new_obj, new_p)


# speedup vs baseline: 1.0716x; 1.0716x over previous
import functools

import jax
import jax.numpy as jnp
from jax import lax
from jax.experimental import pallas as pl
from jax.experimental.pallas import tpu as pltpu
from jax.experimental.pallas import tpu_sc as plsc

O = 10000
E = 160000
DIN = 128
H = 128
DOUT = 128
EPS = 1e-3

NC = 2
NS = 16
NW = NC * NS
LANES = 16

CHUNK_G = 64
NCHUNK_G = E // CHUNK_G
BASE_G = NCHUNK_G // NW
EXTRA_G = NCHUNK_G - BASE_G * NW
CHUNK_S = 32
ROWS_S = 2 * CHUNK_S
NCHUNK_S = E // CHUNK_S
BASE_S = NCHUNK_S // NW
EXTRA_S = NCHUNK_S - BASE_S * NW
HP = H + LANES
O_PAD = 10240
ROWS_PER_SUB = O_PAD // NS

@functools.cache
def _sc_mesh():
  return plsc.VectorSubcoreMesh(
      core_axis_name="c", subcore_axis_name="s", num_cores=NC,
      num_subcores=NS)


def _lrelu(x):
  return jnp.where(x >= 0, x, 0.2 * x)


def _proj_nodes_body(obj_ref, wa_ref, wc_ref, pa_ref, pc_ref):
  x = obj_ref[...]
  pa_ref[...] = jnp.dot(x, wa_ref[...], preferred_element_type=jnp.float32)
  pc_ref[...] = jnp.dot(x, wc_ref[...], preferred_element_type=jnp.float32)


def _proj_nodes(obj_vecs, w1a, w1c):
  bo = 5000
  return pl.pallas_call(
      _proj_nodes_body,
      grid=(O // bo,),
      in_specs=[
          pl.BlockSpec((bo, DIN), lambda i: (i, 0)),
          pl.BlockSpec((DIN, H), lambda i: (0, 0)),
          pl.BlockSpec((DIN, H), lambda i: (0, 0)),
      ],
      out_specs=[
          pl.BlockSpec((bo, H), lambda i: (i, 0)),
          pl.BlockSpec((bo, H), lambda i: (i, 0)),
      ],
      out_shape=[
          jax.ShapeDtypeStruct((O, H), jnp.float32),
          jax.ShapeDtypeStruct((O, H), jnp.float32),
      ],
  )(obj_vecs, w1a, w1c)


def _proj_pred_body(pred_ref, wb_ref, b1_ref, out_ref):
  out_ref[...] = (
      jnp.dot(pred_ref[...], wb_ref[...], preferred_element_type=jnp.float32)
      + b1_ref[...])


def _proj_pred(pred_vecs, w1b, b1r):
  be = 4000
  return pl.pallas_call(
      _proj_pred_body,
      grid=(E // be,),
      in_specs=[
          pl.BlockSpec((be, DIN), lambda i: (i, 0)),
          pl.BlockSpec((DIN, H), lambda i: (0, 0)),
          pl.BlockSpec((1, H), lambda i: (0, 0)),
      ],
      out_specs=pl.BlockSpec((be, H), lambda i: (i, 0)),
      out_shape=jax.ShapeDtypeStruct((E, H), jnp.float32),
  )(pred_vecs, w1b, b1r)


def _sc_gather_body(pa_hbm, pc_hbm, ppred_hbm, sidx_hbm, oidx_hbm,
                    hpre_hbm, stats_hbm,
                    idx_s, idx_o, buf_s, buf_o, buf_p, stat_v, sem):
  c = lax.axis_index("c")
  s = lax.axis_index("s")
  wid = c * NS + s
  trips = BASE_G + jnp.where(wid < EXTRA_G, 1, 0)

  zero = jnp.zeros((LANES,), jnp.float32)
  acc0 = (zero,) * 8 + (zero,) * 8

  def chunk_body(i, acc):
    base = (i * NW + wid) * CHUNK_G
    pltpu.sync_copy(sidx_hbm.at[pl.ds(base, CHUNK_G)], idx_s)
    pltpu.sync_copy(oidx_hbm.at[pl.ds(base, CHUNK_G)], idx_o)
    cp_s = pltpu.async_copy(pa_hbm.at[idx_s], buf_s, sem)
    cp_o = pltpu.async_copy(pc_hbm.at[idx_o], buf_o, sem)
    pltpu.sync_copy(ppred_hbm.at[pl.ds(base, CHUNK_G)], buf_p)
    cp_s.wait()
    cp_o.wait()

    def row_body(r, a):
      new = list(a)
      for j in range(8):
        sl = pl.ds(j * LANES, LANES)
        v = buf_s[r, sl] + buf_o[r, sl] + buf_p[r, sl]
        buf_p[r, sl] = v
        new[j] = a[j] + v
        new[8 + j] = a[8 + j] + v * v
      return tuple(new)

    acc = lax.fori_loop(0, CHUNK_G, row_body, acc)
    pltpu.sync_copy(buf_p, hpre_hbm.at[pl.ds(base, CHUNK_G)])
    return acc

  acc = lax.fori_loop(0, trips, chunk_body, acc0)
  for j in range(8):
    sl = pl.ds(j * LANES, LANES)
    stat_v[0, sl] = acc[j]
    stat_v[1, sl] = acc[8 + j]
  pltpu.sync_copy(stat_v, stats_hbm.at[wid])


@functools.cache
def _sc_gather_kernel():
  return pl.kernel(
      _sc_gather_body,
      out_type=(
          jax.ShapeDtypeStruct((E, H), jnp.float32),
          jax.ShapeDtypeStruct((NW, 8, H), jnp.float32),
      ),
      mesh=_sc_mesh(),
      scratch_types=[
          pltpu.VMEM((CHUNK_G,), jnp.int32),
          pltpu.VMEM((CHUNK_G,), jnp.int32),
          pltpu.VMEM((CHUNK_G, H), jnp.float32),
          pltpu.VMEM((CHUNK_G, H), jnp.float32),
          pltpu.VMEM((CHUNK_G, H), jnp.float32),
          pltpu.VMEM((8, H), jnp.float32),
          pltpu.SemaphoreType.DMA,
      ],
  )


def _sc_gather(*args):
  return _sc_gather_kernel()(*args)


def _bn1_affine_body(stats_ref, g1_ref, be1_ref, scale_ref, shift_ref):
  s1 = jnp.sum(stats_ref[:, 0, :], axis=0, keepdims=True)
  q1 = jnp.sum(stats_ref[:, 1, :], axis=0, keepdims=True)
  m = s1 / E
  v = q1 / E - m * m
  scale = g1_ref[...] * lax.rsqrt(v + EPS)
  scale_ref[...] = scale
  shift_ref[...] = be1_ref[...] - m * scale


def _bn1_affine(stats, g1r, be1r):
  return pl.pallas_call(
      _bn1_affine_body,
      out_shape=[
          jax.ShapeDtypeStruct((1, H), jnp.float32),
          jax.ShapeDtypeStruct((1, H), jnp.float32),
      ],
  )(stats, g1r, be1r)


def _gram_body(hpre_ref, scale_ref, shift_ref, sum_ref, gram_ref,
               acc_s, acc_g):
  i = pl.program_id(0)

  @pl.when(i == 0)
  def _():
    acc_s[...] = jnp.zeros_like(acc_s)
    acc_g[...] = jnp.zeros_like(acc_g)

  h = _lrelu(hpre_ref[...] * scale_ref[...] + shift_ref[...])
  acc_s[...] += jnp.sum(h, axis=0, keepdims=True)
  acc_g[...] += lax.dot_general(h, h, (((0,), (0,)), ((), ())),
                                preferred_element_type=jnp.float32)

  @pl.when(i == pl.num_programs(0) - 1)
  def _():
    sum_ref[...] = acc_s[...]
    gram_ref[...] = acc_g[...]


def _gram(hpre, scale1, shift1):
  be = 4000
  return pl.pallas_call(
      _gram_body,
      grid=(E // be,),
      in_specs=[
          pl.BlockSpec((be, H), lambda i: (i, 0)),
          pl.BlockSpec((1, H), lambda i: (0, 0)),
          pl.BlockSpec((1, H), lambda i: (0, 0)),
      ],
      out_specs=[
          pl.BlockSpec((1, H), lambda i: (0, 0)),
          pl.BlockSpec((H, H), lambda i: (0, 0)),
      ],
      out_shape=[
          jax.ShapeDtypeStruct((1, H), jnp.float32),
          jax.ShapeDtypeStruct((H, H), jnp.float32),
      ],
      scratch_shapes=[
          pltpu.VMEM((1, H), jnp.float32),
          pltpu.VMEM((H, H), jnp.float32),
      ],
  )(hpre, scale1, shift1)


def _bn2_affine_body(sum_ref, gram_ref, w2_ref, b2_ref, g2_ref, be2_ref,
                     scale_ref, shift_ref):
  mh = sum_ref[...] / E
  cov = gram_ref[...] / E - mh.T * mh
  w2 = w2_ref[...]
  cw = jnp.dot(cov, w2, preferred_element_type=jnp.float32)
  v2 = jnp.sum(w2 * cw, axis=0, keepdims=True)
  m2 = jnp.dot(mh, w2, preferred_element_type=jnp.float32) + b2_ref[...]
  scale = g2_ref[...] * lax.rsqrt(v2 + EPS)
  scale_ref[...] = scale
  shift_ref[...] = be2_ref[...] - m2 * scale


def _bn2_affine(sumh, gram, w2, b2r, g2r, be2r):
  w = 2 * H + DOUT
  return pl.pallas_call(
      _bn2_affine_body,
      out_shape=[
          jax.ShapeDtypeStruct((1, w), jnp.float32),
          jax.ShapeDtypeStruct((1, w), jnp.float32),
      ],
  )(sumh, gram, w2, b2r, g2r, be2r)


def _edge_out_body(hpre_ref, sc1_ref, sh1_ref, w2_ref, sc2_ref, sh2_ref,
                   pay_ref, np_ref):
  h = _lrelu(hpre_ref[...] * sc1_ref[...] + sh1_ref[...])
  t2 = jnp.dot(h, w2_ref[...], preferred_element_type=jnp.float32)
  u = _lrelu(t2 * sc2_ref[...] + sh2_ref[...])
  n = u.shape[0]
  ones = jnp.ones((n, LANES), jnp.float32)
  s_rows = jnp.concatenate([u[:, :H], ones], axis=1)
  o_rows = jnp.concatenate([u[:, H + DOUT:], ones], axis=1)
  nb = n // CHUNK_S
  pay = jnp.concatenate([s_rows.reshape(nb, CHUNK_S, HP),
                         o_rows.reshape(nb, CHUNK_S, HP)], axis=1)
  pay_ref[...] = pay
  np_ref[...] = u[:, H:H + DOUT]


def _edge_out(hpre, scale1, shift1, w2, scale2, shift2):
  be = 3200
  nb = be // CHUNK_S
  w = 2 * H + DOUT
  return pl.pallas_call(
      _edge_out_body,
      grid=(E // be,),
      in_specs=[
          pl.BlockSpec((be, H), lambda i: (i, 0)),
          pl.BlockSpec((1, H), lambda i: (0, 0)),
          pl.BlockSpec((1, H), lambda i: (0, 0)),
          pl.BlockSpec((H, w), lambda i: (0, 0)),
          pl.BlockSpec((1, w), lambda i: (0, 0)),
          pl.BlockSpec((1, w), lambda i: (0, 0)),
      ],
      out_specs=[
          pl.BlockSpec((nb, ROWS_S, HP), lambda i: (i, 0, 0)),
          pl.BlockSpec((be, DOUT), lambda i: (i, 0)),
      ],
      out_shape=[
          jax.ShapeDtypeStruct((NCHUNK_S, ROWS_S, HP), jnp.float32),
          jax.ShapeDtypeStruct((E, DOUT), jnp.float32),
      ],
  )(hpre, scale1, shift1, w2, scale2, shift2)


def _sc_scatter_body(pay_hbm, midx_hbm,
                     pooled_hbm,
                     pooled_sh, idx_v, buf_v, sem_d, sem_sc):
  c = lax.axis_index("c")
  s = lax.axis_index("s")
  wid = c * NS + s
  trips = BASE_S + jnp.where(wid < EXTRA_S, 1, 0)

  zero16 = jnp.zeros((LANES,), jnp.float32)

  def zbuf_body(r, _):
    for j in range(HP // LANES):
      buf_v[r, pl.ds(j * LANES, LANES)] = zero16
    return 0

  lax.fori_loop(0, ROWS_S, zbuf_body, 0)

  n_blk = ROWS_PER_SUB // ROWS_S
  row0 = s * ROWS_PER_SUB

  def zero_body(t, _):
    pltpu.sync_copy(buf_v, pooled_sh.at[pl.ds(row0 + t * ROWS_S, ROWS_S)])
    return 0

  lax.fori_loop(0, n_blk, zero_body, 0)
  plsc.subcore_barrier()

  def chunk_body(i, _):
    g = i * NW + wid
    pltpu.sync_copy(midx_hbm.at[g], idx_v)
    pltpu.sync_copy(pay_hbm.at[g], buf_v)
    pltpu.sync_copy(buf_v, pooled_sh.at[idx_v], add=True)
    return 0

  lax.fori_loop(0, trips, chunk_body, 0)
  plsc.subcore_barrier()

  def dump_body(t, _):
    sl = pl.ds(row0 + t * ROWS_S, ROWS_S)
    pltpu.sync_copy(pooled_sh.at[sl], buf_v)
    pltpu.sync_copy(buf_v, pooled_hbm.at[c, sl])
    return 0

  lax.fori_loop(0, n_blk, dump_body, 0)


@functools.cache
def _sc_scatter_kernel():
  return pl.kernel(
      _sc_scatter_body,
      out_type=jax.ShapeDtypeStruct((NC, O_PAD, HP), jnp.float32),
      mesh=_sc_mesh(),
      compiler_params=pltpu.CompilerParams(use_tc_tiling_on_sc=False),
      scratch_types=[
          pltpu.VMEM_SHARED((O_PAD, HP), jnp.float32),
          pltpu.VMEM((ROWS_S,), jnp.int32),
          pltpu.VMEM((ROWS_S, HP), jnp.float32),
          pltpu.SemaphoreType.DMA,
          pltpu.SemaphoreType.DMA,
      ],
  )


def _sc_scatter(*args):
  return _sc_scatter_kernel()(*args)


def _node_mlp_body(pp_ref, w3_ref, b3_ref, g3_ref, be3_ref,
                   w4_ref, b4_ref, g4_ref, be4_ref, out_ref):
  acc = pp_ref[0, :O, :] + pp_ref[1, :O, :]
  pooled = acc[:, :H]
  cnt = jnp.clip(acc[:, H:H + 1], 1.0, float(O))
  pooled = pooled / cnt

  x = jnp.dot(pooled, w3_ref[...], preferred_element_type=jnp.float32)
  x = x + b3_ref[...]
  m = jnp.mean(x, axis=0, keepdims=True)
  v = jnp.mean((x - m) * (x - m), axis=0, keepdims=True)
  h2 = _lrelu((x - m) * lax.rsqrt(v + EPS) * g3_ref[...] + be3_ref[...])

  y = jnp.dot(h2, w4_ref[...], preferred_element_type=jnp.float32)
  y = y + b4_ref[...]
  m2 = jnp.mean(y, axis=0, keepdims=True)
  v2 = jnp.mean((y - m2) * (y - m2), axis=0, keepdims=True)
  out_ref[...] = _lrelu((y - m2) * lax.rsqrt(v2 + EPS) * g4_ref[...]
                        + be4_ref[...])


def _node_mlp(pooledp, w3, b3r, g3r, be3r, w4, b4r, g4r, be4r):
  return pl.pallas_call(
      _node_mlp_body,
      out_shape=jax.ShapeDtypeStruct((O, DOUT), jnp.float32),
  )(pooledp, w3, b3r, g3r, be3r, w4, b4r, g4r, be4r)


def kernel(obj_vecs, pred_vecs, edges, W1, b1, g1, be1, W2, b2, g2, be2,
           W3, b3, g3, be3, W4, b4, g4, be4):
  edges = edges.astype(jnp.int32)
  s_idx = edges[:, 0]
  o_idx = edges[:, 1]

  w1a = W1[0:DIN, :]
  w1b = W1[DIN:2 * DIN, :]
  w1c = W1[2 * DIN:3 * DIN, :]

  b1r = b1.reshape(1, H)
  g1r = g1.reshape(1, H)
  be1r = be1.reshape(1, H)
  w = 2 * H + DOUT
  b2r = b2.reshape(1, w)
  g2r = g2.reshape(1, w)
  be2r = be2.reshape(1, w)
  b3r = b3.reshape(1, H)
  g3r = g3.reshape(1, H)
  be3r = be3.reshape(1, H)
  b4r = b4.reshape(1, DOUT)
  g4r = g4.reshape(1, DOUT)
  be4r = be4.reshape(1, DOUT)

  pa, pc = _proj_nodes(obj_vecs, w1a, w1c)
  ppred = _proj_pred(pred_vecs, w1b, b1r)

  hpre, stats = _sc_gather(pa, pc, ppred, s_idx, o_idx)

  scale1, shift1 = _bn1_affine(stats, g1r, be1r)
  sumh, gram = _gram(hpre, scale1, shift1)
  scale2, shift2 = _bn2_affine(sumh, gram, W2, b2r, g2r, be2r)

  payload, new_p = _edge_out(hpre, scale1, shift1, W2, scale2, shift2)
  midx = jnp.concatenate([s_idx.reshape(NCHUNK_S, CHUNK_S),
                          o_idx.reshape(NCHUNK_S, CHUNK_S)], axis=1)

  pooledp = _sc_scatter(payload, midx)

  new_obj = _node_mlp(pooledp, W3, b3r, g3r, be3r, W4, b4r, g4r, be4r)
  return (new_obj, new_p)
